# Initial kernel scaffold; baseline (speedup 1.0000x reference)
import jax
import jax.numpy as jnp
from jax.experimental import pallas as pl


def kernel(x, edge_index, batch, current_y, W1, b1, W2, b2, L1w, L1b, L2w, L2b):
    raise NotImplementedError("write your pallas kernel here")


# speedup vs baseline: 12.2292x; 12.2292x over previous
import functools

import jax
import jax.numpy as jnp
from jax import lax
from jax.experimental import pallas as pl
from jax.experimental.pallas import tpu as pltpu
from jax.experimental.pallas import tpu_sc as plsc

N = 10000
F = 128
K = 100
C = 10
E = 320000
NPAD = 10112
DP = 128
NC, NS = 2, 16
NW = NC * NS
CH = 128
NCHK = 79
EPW = NCHK * CH
EPAD = NW * EPW
RPS = NPAD // NS

def _dot(a, b, dims):
    return lax.dot_general(a, b, (dims, ((), ())),
                           preferred_element_type=jnp.float32)


def _sc_degree_body(dst_hbm, zdp_hbm, ones_hbm, out_hbm, idx_v, ones_v, acc):
    c = lax.axis_index("c")
    s = lax.axis_index("s")
    wid = c * NS + s
    pltpu.sync_copy(zdp_hbm.at[pl.ds(s * RPS, RPS)], acc.at[pl.ds(s * RPS, RPS)])
    pltpu.sync_copy(dst_hbm.at[wid], idx_v)
    pltpu.sync_copy(ones_hbm, ones_v)
    plsc.subcore_barrier()

    def body(j, carry):
        pltpu.sync_copy(ones_v, acc.at[idx_v.at[j]], add=True)
        return carry

    lax.fori_loop(0, NCHK, body, 0)
    plsc.subcore_barrier()
    pltpu.sync_copy(acc.at[pl.ds(s * RPS, RPS)], out_hbm.at[c, pl.ds(s * RPS, RPS)])


def _sc_aggregate_body(u_hbm, src_hbm, dst_hbm, zdp_hbm, out_hbm,
                       src_v, dst_v, rows_v, acc, sem):
    c = lax.axis_index("c")
    s = lax.axis_index("s")
    wid = c * NS + s
    pltpu.sync_copy(zdp_hbm.at[pl.ds(s * RPS, RPS)], acc.at[pl.ds(s * RPS, RPS)])
    pltpu.sync_copy(src_hbm.at[wid], src_v)
    pltpu.sync_copy(dst_hbm.at[wid], dst_v)
    plsc.subcore_barrier()

    def body(j, carry):
        pltpu.async_copy(u_hbm.at[src_v.at[j]], rows_v, sem).wait()
        pltpu.sync_copy(rows_v, acc.at[dst_v.at[j]], add=True)
        return carry

    lax.fori_loop(0, NCHK, body, 0)
    plsc.subcore_barrier()
    pltpu.sync_copy(acc.at[pl.ds(s * RPS, RPS)], out_hbm.at[c, pl.ds(s * RPS, RPS)])


@functools.cache
def _sc_kernels():
    mesh = plsc.VectorSubcoreMesh(core_axis_name="c", subcore_axis_name="s",
                                  num_cores=NC, num_subcores=NS)
    degree = pl.kernel(
        _sc_degree_body,
        out_type=jax.ShapeDtypeStruct((NC, NPAD, DP), jnp.float32),
        mesh=mesh,
        scratch_types=[
            pltpu.VMEM((NCHK, CH), jnp.int32),
            pltpu.VMEM((CH, DP), jnp.float32),
            pltpu.VMEM_SHARED((NPAD, DP), jnp.float32),
        ],
    )
    aggregate = pl.kernel(
        _sc_aggregate_body,
        out_type=jax.ShapeDtypeStruct((NC, NPAD, DP), jnp.float32),
        mesh=mesh,
        scratch_types=[
            pltpu.VMEM((NCHK, CH), jnp.int32),
            pltpu.VMEM((NCHK, CH), jnp.int32),
            pltpu.VMEM((CH, DP), jnp.float32),
            pltpu.VMEM_SHARED((NPAD, DP), jnp.float32),
            pltpu.SemaphoreType.DMA,
        ],
    )
    return degree, aggregate


def _tc1a_body(x_ref, y_ref, w1_ref, w1eff_ref):
    x = x_ref[...]
    y = y_ref[...]
    lane = lax.broadcasted_iota(jnp.int32, (NPAD, F), 1)
    Y = (y == lane).astype(jnp.float32)
    obs = _dot(Y, x, ((0,), (0,)))
    ones_col = jnp.ones((NPAD, 1), jnp.float32)
    ccol = _dot(Y, ones_col, ((0,), (0,)))
    fc = jnp.sum(x, axis=0, keepdims=True)
    expected = (ccol * (1.0 / N)) * fc
    term = jnp.where(expected > 0.0,
                     (obs - expected) ** 2 / jnp.where(expected > 0.0, expected, 1.0),
                     0.0)
    chi2r = jnp.sum(term, axis=0, keepdims=True)
    ident = (lax.broadcasted_iota(jnp.int32, (F, F), 0)
             == lax.broadcasted_iota(jnp.int32, (F, F), 1)).astype(jnp.float32)
    chi2c = _dot(ident, chi2r, ((1,), (1,)))
    ir = lax.broadcasted_iota(jnp.int32, (F, F), 0)
    ic = lax.broadcasted_iota(jnp.int32, (F, F), 1)
    gt = (chi2c > chi2r).astype(jnp.float32)
    tie = ((chi2c == chi2r) & (ir < ic)).astype(jnp.float32)
    rankr = jnp.sum(gt + tie, axis=0, keepdims=True)
    selr = (rankr < float(K)).astype(jnp.float32)
    selc = _dot(ident, selr, ((1,), (1,)))
    lt = (ic < ir).astype(jnp.float32)
    posc = _dot(lt, selc, ((1,), (0,)))
    lane_f = lax.broadcasted_iota(jnp.int32, (F, F), 1).astype(jnp.float32)
    S = selc * (posc == lane_f).astype(jnp.float32)
    w1eff_ref[...] = _dot(S, w1_ref[...], ((1,), (0,)))


def _tc1b_body(x_ref, w1eff_ref, degp_ref, u1_ref, dinv_ref):
    xw1 = _dot(x_ref[...], w1eff_ref[...], ((1,), (0,)))
    degp = degp_ref[...]
    deg = (degp[0] + degp[1])[:, 0:1] + 1.0
    dinv = lax.rsqrt(deg)
    u1_ref[...] = dinv * xw1
    dinv_ref[...] = dinv


def _tc2_body(part_ref, u_ref, dinv_ref, w2_ref, b1_ref, u2_ref):
    part = part_ref[...]
    dinv = dinv_ref[...]
    h1 = dinv * (part[0] + part[1] + u_ref[...]) + b1_ref[...]
    u2_ref[...] = dinv * _dot(h1, w2_ref[...], ((1,), (0,)))


def _tc3_body(part_ref, u_ref, dinv_ref, b2_ref, l1w_ref, l1b_ref,
              l2w_ref, l2b_ref, out_ref):
    part = part_ref[...]
    dinv = dinv_ref[...]
    h2 = dinv * (part[0] + part[1] + u_ref[...]) + b2_ref[...]
    z = jnp.maximum(_dot(h2, l1w_ref[...], ((1,), (0,))) + l1b_ref[...], 0.0)
    out_ref[...] = _dot(l2w_ref[...], z, ((0,), (1,))) + l2b_ref[...]


_tc1a = pl.pallas_call(
    _tc1a_body,
    out_shape=jax.ShapeDtypeStruct((F, DP), jnp.float32),
)

_tc1b = pl.pallas_call(
    _tc1b_body,
    out_shape=(jax.ShapeDtypeStruct((NPAD, DP), jnp.float32),
               jax.ShapeDtypeStruct((NPAD, 1), jnp.float32)),
)

_tc2 = pl.pallas_call(
    _tc2_body,
    out_shape=jax.ShapeDtypeStruct((NPAD, DP), jnp.float32),
)

_tc3 = pl.pallas_call(
    _tc3_body,
    out_shape=jax.ShapeDtypeStruct((1, NPAD), jnp.float32),
)


def kernel(x, edge_index, batch, current_y, W1, b1, W2, b2, L1w, L1b, L2w, L2b):
    f32 = jnp.float32
    xp = jnp.pad(x, ((0, NPAD - N), (0, 0)))
    yp = jnp.pad(current_y.reshape(N, 1), ((0, NPAD - N), (0, 0)),
                 constant_values=-1)
    pad_e = jnp.full((EPAD - E,), N, jnp.int32)
    src = jnp.concatenate([edge_index[0], pad_e]).reshape(NW, NCHK, CH)
    dst = jnp.concatenate([edge_index[1], pad_e]).reshape(NW, NCHK, CH)
    zdp = jnp.zeros((NPAD, DP), f32)
    onesdp = jnp.ones((CH, DP), f32)
    W1p = jnp.pad(W1, ((0, DP - K), (0, DP - 100)))
    W2p = jnp.pad(W2, ((0, DP - 100), (0, DP - 100)))
    b1p = jnp.pad(b1, (0, DP - 100)).reshape(1, DP)
    b2p = jnp.pad(b2, (0, DP - 100)).reshape(1, DP)
    L1wp = jnp.pad(L1w, ((0, DP - 100), (0, DP - 10)))
    L1bp = jnp.pad(L1b, (0, DP - 10)).reshape(1, DP)
    L2wp = jnp.pad(L2w, ((0, DP - 10), (0, 0)))
    L2b11 = L2b.reshape(1, 1)

    sc_degree, sc_aggregate = _sc_kernels()
    degp = sc_degree(dst, zdp, onesdp)
    w1eff = _tc1a(xp, yp, W1p)
    u1, dinv = _tc1b(xp, w1eff, degp)
    part1 = sc_aggregate(u1, src, dst, zdp)
    u2 = _tc2(part1, u1, dinv, W2p, b1p)
    part2 = sc_aggregate(u2, src, dst, zdp)
    outp = _tc3(part2, u2, dinv, b2p, L1wp, L1bp, L2wp, L2b11)
    return outp[:, :N]
